# baseline (speedup 1.0000x reference)
import functools

import jax
import jax.numpy as jnp
from jax import lax
from jax.experimental import pallas as pl
from jax.experimental.pallas import tpu as pltpu
from jax.experimental.pallas import tpu_sc as plsc

E = 8
D = 1024
DFF = 4096
T = 4096
K = 512
EPAD = 128


def _router_topk_body(x_ref, wr_ref, pt_ref, logits_ref, wts_ref, sel_ref):
    logits_ref[...] = lax.dot_general(
        x_ref[...], wr_ref[...], (((1,), (1,)), ((), ())),
        preferred_element_type=jnp.float32)

    v = pt_ref[...]
    idx = lax.broadcasted_iota(jnp.int32, (E, T), 1)
    iot = idx

    k = 2
    while k <= T:
        j = k // 2
        while j >= 1:
            bit = (iot & j) != 0
            pv = jnp.where(bit, jnp.roll(v, j, 1), jnp.roll(v, -j, 1))
            pi = jnp.where(bit, jnp.roll(idx, j, 1), jnp.roll(idx, -j, 1))
            less = (v > pv) | ((v == pv) & (idx < pi))
            want_min = ((iot & k) == 0) ^ bit
            keep = want_min == less
            v = jnp.where(keep, v, pv)
            idx = jnp.where(keep, idx, pi)
            j //= 2
        k *= 2

    wts_ref[...] = v[:, :K]
    sel_ref[...] = idx[:, :K]


def _router_topk(x2d, wr_pad, probs_t):
    return pl.pallas_call(
        _router_topk_body,
        out_shape=(
            jax.ShapeDtypeStruct((T, EPAD), jnp.float32),
            jax.ShapeDtypeStruct((E, K), jnp.float32),
            jax.ShapeDtypeStruct((E, K), jnp.int32),
        ),
    )(x2d, wr_pad, probs_t)


NF = 8
F = DFF // NF


def _erf(x):
    s = jnp.sign(x)
    a = jnp.abs(x)
    t = 1.0 / (1.0 + 0.3275911 * a)
    p = ((((1.061405429 * t - 1.453152027) * t + 1.421413741) * t
          - 0.284496736) * t + 0.254829592) * t
    return s * (1.0 - p * jnp.exp(-a * a))


def _gelu(x):
    return 0.5 * x * (1.0 + _erf(x * 0.7071067811865476))


def _mlp_body(xs_ref, w1_ref, b1_ref, w2_ref, b2_ref, wt_ref, out_ref, acc_ref):
    f = pl.program_id(1)
    x2 = xs_ref[0].astype(jnp.bfloat16)
    h = lax.dot_general(x2, w1_ref[0], (((1,), (1,)), ((), ())),
                        preferred_element_type=jnp.float32)
    h = _gelu(h + b1_ref[0])
    part = lax.dot_general(h.astype(jnp.bfloat16), w2_ref[0],
                           (((1,), (1,)), ((), ())),
                           preferred_element_type=jnp.float32)

    @pl.when(f == 0)
    def _():
        acc_ref[...] = part

    @pl.when(f > 0)
    def _():
        acc_ref[...] = acc_ref[...] + part

    @pl.when(f == NF - 1)
    def _():
        w = wt_ref[0][:, :1]
        out_ref[0] = (acc_ref[...] + b2_ref[0]) * w


def _mlp(xs3, W1, b1, W2, b2, wts_b):
    return pl.pallas_call(
        _mlp_body,
        grid=(E, NF),
        in_specs=[
            pl.BlockSpec((1, K, D), lambda e, f: (e, 0, 0)),
            pl.BlockSpec((1, F, D), lambda e, f: (e, f, 0)),
            pl.BlockSpec((1, 1, F), lambda e, f: (e, 0, f)),
            pl.BlockSpec((1, D, F), lambda e, f: (e, 0, f)),
            pl.BlockSpec((1, 1, D), lambda e, f: (e, 0, 0)),
            pl.BlockSpec((1, K, 128), lambda e, f: (e, 0, 0)),
        ],
        out_specs=pl.BlockSpec((1, K, D), lambda e, f: (e, 0, 0)),
        out_shape=jax.ShapeDtypeStruct((E, K, D), jnp.float32),
        scratch_shapes=[pltpu.VMEM((K, D), jnp.float32)],
    )(xs3, W1.astype(jnp.bfloat16), b1[:, None, :],
      W2.astype(jnp.bfloat16), b2[:, None, :], wts_b)


NC = 2
NS = 16
NW = NC * NS
GROWS = T // NW
GCHUNK = 32
GNCH = GROWS // GCHUNK


@functools.lru_cache(maxsize=None)
def _make_sc_gather():
    @functools.partial(
        pl.kernel,
        out_type=jax.ShapeDtypeStruct((T, D), jnp.float32),
        mesh=plsc.VectorSubcoreMesh(core_axis_name="c", subcore_axis_name="s",
                                    num_cores=NC, num_subcores=NS),
        scratch_types=[
            pltpu.VMEM((GCHUNK,), jnp.int32),
            pltpu.VMEM((GCHUNK, D), jnp.float32),
            pltpu.SemaphoreType.DMA,
        ],
        compiler_params=pltpu.CompilerParams(needs_layout_passes=False),
    )
    def _sc_gather(x_hbm, sel_hbm, out_hbm, idx_v, rows_v, sem):
        wid = lax.axis_index("s") * NC + lax.axis_index("c")
        base = wid * GROWS
        for c in range(GNCH):
            off = base + c * GCHUNK
            pltpu.sync_copy(sel_hbm.at[pl.ds(off, GCHUNK)], idx_v)
            pltpu.async_copy(x_hbm.at[idx_v], rows_v, sem).wait()
            pltpu.sync_copy(rows_v, out_hbm.at[pl.ds(off, GCHUNK)])

    return _sc_gather


TOK_B = 64
NBUK = T // TOK_B
CAPM = 544
SPAD = T + 32


@functools.lru_cache(maxsize=None)
def _make_sc_scatter():
    @functools.partial(
        pl.kernel,
        out_type=jax.ShapeDtypeStruct((T, D), jnp.float32),
        mesh=plsc.VectorSubcoreMesh(core_axis_name="c", subcore_axis_name="s",
                                    num_cores=NC, num_subcores=NS),
        scratch_types=[
            pltpu.VMEM((T,), jnp.int32),
            pltpu.VMEM((CAPM,), jnp.int32),
            pltpu.VMEM((CAPM,), jnp.int32),
            pltpu.VMEM((CAPM // 32, 32), jnp.int32),
            pltpu.VMEM((CAPM // 32, 32), jnp.int32),
            pltpu.VMEM((32, D), jnp.float32),
            pltpu.VMEM((TOK_B, D), jnp.float32),
            pltpu.SemaphoreType.DMA,
        ],
        compiler_params=pltpu.CompilerParams(needs_layout_passes=False),
    )
    def _sc_scatter(src_hbm, sel_hbm, zeros_hbm, out_hbm,
                    sel_v, mlist, llist, m2, l2, buf, acc, sem):
        wid = lax.axis_index("s") * NC + lax.axis_index("c")
        pltpu.sync_copy(sel_hbm, sel_v)
        dummy_slot = jnp.full((16,), T, jnp.int32)
        zero16 = jnp.zeros((16,), jnp.int32)
        for p in range(NBUK // NW):
            tb = (p * NW + wid) * TOK_B
            pltpu.sync_copy(zeros_hbm, acc)

            def fill(i, _):
                mlist[pl.ds(i * 16, 16)] = dummy_slot
                llist[pl.ds(i * 16, 16)] = zero16
                return 0

            lax.fori_loop(0, CAPM // 16, fill, 0)

            lane16 = lax.iota(jnp.int32, 16)

            def scan(g, cnt):
                tk = sel_v[pl.ds(g * 16, 16)]
                loc = tk - tb
                m = (loc >= 0) & (loc < TOK_B)
                slots = lane16 + g * 16
                ones = jnp.where(m, 1, 0)
                pos = cnt + plsc.cumsum(ones) - 1
                pos = jnp.where(m, pos, CAPM - 16 + lane16)
                plsc.store_scatter(mlist, [pos], slots)
                plsc.store_scatter(llist, [pos], jnp.where(m, loc, 0))
                return cnt + jnp.sum(ones)

            cnt = lax.fori_loop(0, T // 16, scan, jnp.int32(0))
            nch = (cnt + 31) // 32

            for c in range(CAPM // 32):
                m2[c, pl.ds(0, 16)] = mlist[pl.ds(c * 32, 16)]
                m2[c, pl.ds(16, 16)] = mlist[pl.ds(c * 32 + 16, 16)]
                l2[c, pl.ds(0, 16)] = llist[pl.ds(c * 32, 16)]
                l2[c, pl.ds(16, 16)] = llist[pl.ds(c * 32 + 16, 16)]

            def gath(c, _):
                pltpu.async_copy(src_hbm.at[m2.at[c]], buf, sem).wait()

                def row(r, _):
                    lvec = l2[c, pl.ds((r // 16) * 16, 16)]
                    lr = jnp.sum(jnp.where(lane16 == (r % 16), lvec, 0))

                    def addrow(cc, _):
                        plsc.addupdate(acc.at[lr, pl.ds(cc * 16, 16)],
                                       buf[r, pl.ds(cc * 16, 16)])
                        return 0

                    lax.fori_loop(0, D // 16, addrow, 0)
                    return 0

                lax.fori_loop(0, 32, row, 0)
                return 0

            lax.fori_loop(0, nch, gath, 0)
            pltpu.sync_copy(acc, out_hbm.at[pl.ds(tb, TOK_B)])

    return _sc_scatter


def kernel(inputs, Wr, W1, b1, W2, b2):
    x2d = inputs.reshape(-1, D)

    router_logits_x = x2d @ Wr.T
    probs_t = jax.nn.softmax(router_logits_x.astype(jnp.float32), axis=1).T

    wr_pad = jnp.zeros((EPAD, D), jnp.float32).at[:E].set(Wr)
    logits_pad, wts, sel = _router_topk(x2d, wr_pad, probs_t)
    router_logits = logits_pad[:, :E]

    sel_flat = sel.reshape(T)
    xs = _make_sc_gather()(x2d, sel_flat)

    wts_b = jnp.broadcast_to(wts[:, :, None], (E, K, 128))
    scaled = _mlp(xs.reshape(E, K, D), W1, b1, W2, b2, wts_b)

    zeros_hbm = jnp.zeros((TOK_B, D), jnp.float32)
    scaled_pad = jnp.concatenate(
        [scaled.reshape(T, D), jnp.zeros((SPAD - T, D), jnp.float32)])
    results = _make_sc_scatter()(scaled_pad, sel_flat, zeros_hbm)
    return results.reshape(inputs.shape), router_logits, sel


# speedup vs baseline: 1.3186x; 1.3186x over previous
import functools

import jax
import jax.numpy as jnp
from jax import lax
from jax.experimental import pallas as pl
from jax.experimental.pallas import tpu as pltpu
from jax.experimental.pallas import tpu_sc as plsc

E = 8
D = 1024
DFF = 4096
T = 4096
K = 512
EPAD = 128


def _router_topk_body(x_ref, wr_ref, pt_ref, logits_ref, wts_ref, sel_ref):
    logits_ref[...] = lax.dot_general(
        x_ref[...], wr_ref[...], (((1,), (1,)), ((), ())),
        preferred_element_type=jnp.float32)

    v = pt_ref[...]
    idx = lax.broadcasted_iota(jnp.int32, (E, T), 1)
    iot = idx

    k = 2
    while k <= T:
        j = k // 2
        while j >= 1:
            bit = (iot & j) != 0
            pv = jnp.where(bit, jnp.roll(v, j, 1), jnp.roll(v, -j, 1))
            pi = jnp.where(bit, jnp.roll(idx, j, 1), jnp.roll(idx, -j, 1))
            less = (v > pv) | ((v == pv) & (idx < pi))
            want_min = ((iot & k) == 0) ^ bit
            keep = want_min == less
            v = jnp.where(keep, v, pv)
            idx = jnp.where(keep, idx, pi)
            j //= 2
        k *= 2

    wts_ref[...] = v[:, :K]
    sel_ref[...] = idx[:, :K]


def _router_topk(x2d, wr_pad, probs_t):
    return pl.pallas_call(
        _router_topk_body,
        out_shape=(
            jax.ShapeDtypeStruct((T, EPAD), jnp.float32),
            jax.ShapeDtypeStruct((E, K), jnp.float32),
            jax.ShapeDtypeStruct((E, K), jnp.int32),
        ),
    )(x2d, wr_pad, probs_t)


NF = 8
F = DFF // NF


def _erf(x):
    s = jnp.sign(x)
    a = jnp.abs(x)
    t = 1.0 / (1.0 + 0.3275911 * a)
    p = ((((1.061405429 * t - 1.453152027) * t + 1.421413741) * t
          - 0.284496736) * t + 0.254829592) * t
    return s * (1.0 - p * jnp.exp(-a * a))


def _gelu(x):
    return 0.5 * x * (1.0 + _erf(x * 0.7071067811865476))


def _mlp_body(xs_ref, w1_ref, b1_ref, w2_ref, b2_ref, wt_ref, out_ref, acc_ref):
    f = pl.program_id(1)
    x2 = xs_ref[0].astype(jnp.bfloat16)
    h = lax.dot_general(x2, w1_ref[0].astype(jnp.bfloat16),
                        (((1,), (1,)), ((), ())),
                        preferred_element_type=jnp.float32)
    h = _gelu(h + b1_ref[0])
    part = lax.dot_general(h.astype(jnp.bfloat16), w2_ref[0].astype(jnp.bfloat16),
                           (((1,), (1,)), ((), ())),
                           preferred_element_type=jnp.float32)

    @pl.when(f == 0)
    def _():
        acc_ref[...] = part

    @pl.when(f > 0)
    def _():
        acc_ref[...] = acc_ref[...] + part

    @pl.when(f == NF - 1)
    def _():
        w = wt_ref[0][:, :1]
        out_ref[0] = (acc_ref[...] + b2_ref[0]) * w


def _mlp(xs3, W1, b1, W2, b2, wts_b):
    return pl.pallas_call(
        _mlp_body,
        grid=(E, NF),
        in_specs=[
            pl.BlockSpec((1, K, D), lambda e, f: (e, 0, 0)),
            pl.BlockSpec((1, F, D), lambda e, f: (e, f, 0)),
            pl.BlockSpec((1, 1, F), lambda e, f: (e, 0, f)),
            pl.BlockSpec((1, D, F), lambda e, f: (e, 0, f)),
            pl.BlockSpec((1, 1, D), lambda e, f: (e, 0, 0)),
            pl.BlockSpec((1, K, 128), lambda e, f: (e, 0, 0)),
        ],
        out_specs=pl.BlockSpec((1, K, D), lambda e, f: (e, 0, 0)),
        out_shape=jax.ShapeDtypeStruct((E, K, D), jnp.float32),
        scratch_shapes=[pltpu.VMEM((K, D), jnp.float32)],
    )(xs3, W1, b1[:, None, :], W2, b2[:, None, :], wts_b)


NC = 2
NS = 16
NW = NC * NS
GROWS = T // NW
GCHUNK = 32
GNCH = GROWS // GCHUNK


@functools.lru_cache(maxsize=None)
def _make_sc_gather():
    @functools.partial(
        pl.kernel,
        out_type=jax.ShapeDtypeStruct((T, D), jnp.float32),
        mesh=plsc.VectorSubcoreMesh(core_axis_name="c", subcore_axis_name="s",
                                    num_cores=NC, num_subcores=NS),
        scratch_types=[
            pltpu.VMEM((GCHUNK,), jnp.int32),
            pltpu.VMEM((GCHUNK, D), jnp.float32),
            pltpu.SemaphoreType.DMA,
        ],
        compiler_params=pltpu.CompilerParams(needs_layout_passes=False),
    )
    def _sc_gather(x_hbm, sel_hbm, out_hbm, idx_v, rows_v, sem):
        wid = lax.axis_index("s") * NC + lax.axis_index("c")
        base = wid * GROWS
        for c in range(GNCH):
            off = base + c * GCHUNK
            pltpu.sync_copy(sel_hbm.at[pl.ds(off, GCHUNK)], idx_v)
            pltpu.async_copy(x_hbm.at[idx_v], rows_v, sem).wait()
            pltpu.sync_copy(rows_v, out_hbm.at[pl.ds(off, GCHUNK)])

    return _sc_gather


TOK_B = 64
NBUK = T // TOK_B
CAPM = 544
SPAD = T + 32


@functools.lru_cache(maxsize=None)
def _make_sc_scatter():
    @functools.partial(
        pl.kernel,
        out_type=jax.ShapeDtypeStruct((T, D), jnp.float32),
        mesh=plsc.VectorSubcoreMesh(core_axis_name="c", subcore_axis_name="s",
                                    num_cores=NC, num_subcores=NS),
        scratch_types=[
            pltpu.VMEM((T,), jnp.int32),
            pltpu.VMEM((CAPM,), jnp.int32),
            pltpu.VMEM((CAPM,), jnp.int32),
            pltpu.VMEM((CAPM // 32, 32), jnp.int32),
            pltpu.VMEM((CAPM // 32, 32), jnp.int32),
            pltpu.VMEM((32, D), jnp.float32),
            pltpu.VMEM((TOK_B, D), jnp.float32),
            pltpu.SemaphoreType.DMA,
        ],
        compiler_params=pltpu.CompilerParams(needs_layout_passes=False),
    )
    def _sc_scatter(src_hbm, sel_hbm, zeros_hbm, out_hbm,
                    sel_v, mlist, llist, m2, l2, buf, acc, sem):
        wid = lax.axis_index("s") * NC + lax.axis_index("c")
        pltpu.sync_copy(sel_hbm, sel_v)
        dummy_slot = jnp.full((16,), T, jnp.int32)
        zero16 = jnp.zeros((16,), jnp.int32)
        for p in range(NBUK // NW):
            tb = (p * NW + wid) * TOK_B
            pltpu.sync_copy(zeros_hbm, acc)

            def fill(i, _):
                mlist[pl.ds(i * 16, 16)] = dummy_slot
                llist[pl.ds(i * 16, 16)] = zero16
                return 0

            lax.fori_loop(0, CAPM // 16, fill, 0)

            lane16 = lax.iota(jnp.int32, 16)

            def scan(g, cnt):
                tk = sel_v[pl.ds(g * 16, 16)]
                loc = tk - tb
                m = (loc >= 0) & (loc < TOK_B)
                slots = lane16 + g * 16
                ones = jnp.where(m, 1, 0)
                pos = cnt + plsc.cumsum(ones) - 1
                pos = jnp.where(m, pos, CAPM - 16 + lane16)
                plsc.store_scatter(mlist, [pos], slots)
                plsc.store_scatter(llist, [pos], jnp.where(m, loc, 0))
                return cnt + jnp.sum(ones)

            cnt = lax.fori_loop(0, T // 16, scan, jnp.int32(0))
            nch = (cnt + 31) // 32

            for c in range(CAPM // 32):
                m2[c, pl.ds(0, 16)] = mlist[pl.ds(c * 32, 16)]
                m2[c, pl.ds(16, 16)] = mlist[pl.ds(c * 32 + 16, 16)]
                l2[c, pl.ds(0, 16)] = llist[pl.ds(c * 32, 16)]
                l2[c, pl.ds(16, 16)] = llist[pl.ds(c * 32 + 16, 16)]

            def gath(c, _):
                pltpu.async_copy(src_hbm.at[m2.at[c]], buf, sem).wait()

                def row(r, _):
                    lvec = l2[c, pl.ds((r // 16) * 16, 16)]
                    lr = jnp.sum(jnp.where(lane16 == (r % 16), lvec, 0))

                    def addrow(cc, _):
                        plsc.addupdate(acc.at[lr, pl.ds(cc * 16, 16)],
                                       buf[r, pl.ds(cc * 16, 16)])
                        return 0

                    lax.fori_loop(0, D // 16, addrow, 0)
                    return 0

                lax.fori_loop(0, 32, row, 0)
                return 0

            lax.fori_loop(0, nch, gath, 0)
            pltpu.sync_copy(acc, out_hbm.at[pl.ds(tb, TOK_B)])

    return _sc_scatter


def kernel(inputs, Wr, W1, b1, W2, b2):
    x2d = inputs.reshape(-1, D)

    router_logits_x = x2d @ Wr.T
    probs_t = jax.nn.softmax(router_logits_x.astype(jnp.float32), axis=1).T

    wr_pad = jnp.zeros((EPAD, D), jnp.float32).at[:E].set(Wr)
    logits_pad, wts, sel = _router_topk(x2d, wr_pad, probs_t)
    router_logits = logits_pad[:, :E]

    sel_flat = sel.reshape(T)
    xs = _make_sc_gather()(x2d, sel_flat)

    wts_b = jnp.broadcast_to(wts[:, :, None], (E, K, 128))
    scaled = _mlp(xs.reshape(E, K, D), W1, b1, W2, b2, wts_b)

    zeros_hbm = jnp.zeros((TOK_B, D), jnp.float32)
    scaled_pad = jnp.concatenate(
        [scaled.reshape(T, D), jnp.zeros((SPAD - T, D), jnp.float32)])
    results = _make_sc_scatter()(scaled_pad, sel_flat, zeros_hbm)
    return results.reshape(inputs.shape), router_logits, sel


# speedup vs baseline: 1.4560x; 1.1042x over previous
import functools

import jax
import jax.numpy as jnp
from jax import lax
from jax.experimental import pallas as pl
from jax.experimental.pallas import tpu as pltpu
from jax.experimental.pallas import tpu_sc as plsc

E = 8
D = 1024
DFF = 4096
T = 4096
K = 512
EPAD = 128


def _router_topk_body(x_ref, wr_ref, pt_ref, logits_ref, wts_ref, sel_ref):
    logits_ref[...] = lax.dot_general(
        x_ref[...], wr_ref[...], (((1,), (1,)), ((), ())),
        preferred_element_type=jnp.float32)

    v = pt_ref[...]
    idx = lax.broadcasted_iota(jnp.int32, (E, T), 1)
    iot = idx

    k = 2
    while k <= T:
        j = k // 2
        while j >= 1:
            bit = (iot & j) != 0
            pv = jnp.where(bit, jnp.roll(v, j, 1), jnp.roll(v, -j, 1))
            pi = jnp.where(bit, jnp.roll(idx, j, 1), jnp.roll(idx, -j, 1))
            less = (v > pv) | ((v == pv) & (idx < pi))
            want_min = ((iot & k) == 0) ^ bit
            keep = want_min == less
            v = jnp.where(keep, v, pv)
            idx = jnp.where(keep, idx, pi)
            j //= 2
        k *= 2

    wts_ref[...] = v[:, :K]
    sel_ref[...] = idx[:, :K]


def _router_topk(x2d, wr_pad, probs_t):
    return pl.pallas_call(
        _router_topk_body,
        out_shape=(
            jax.ShapeDtypeStruct((T, EPAD), jnp.float32),
            jax.ShapeDtypeStruct((E, K), jnp.float32),
            jax.ShapeDtypeStruct((E, K), jnp.int32),
        ),
    )(x2d, wr_pad, probs_t)


NF = 4
F = DFF // NF


def _erf(x):
    s = jnp.sign(x)
    a = jnp.abs(x)
    t = 1.0 / (1.0 + 0.3275911 * a)
    p = ((((1.061405429 * t - 1.453152027) * t + 1.421413741) * t
          - 0.284496736) * t + 0.254829592) * t
    return s * (1.0 - p * jnp.exp(-a * a))


def _gelu(x):
    return 0.5 * x * (1.0 + _erf(x * 0.7071067811865476))


def _mlp_body(xs_ref, w1_ref, b1_ref, w2_ref, b2_ref, wt_ref, out_ref, acc_ref):
    f = pl.program_id(1)
    x2 = xs_ref[0].astype(jnp.bfloat16)
    h = lax.dot_general(x2, w1_ref[0].astype(jnp.bfloat16),
                        (((1,), (1,)), ((), ())),
                        preferred_element_type=jnp.float32)
    h = _gelu(h + b1_ref[0])
    part = lax.dot_general(h.astype(jnp.bfloat16), w2_ref[0].astype(jnp.bfloat16),
                           (((1,), (1,)), ((), ())),
                           preferred_element_type=jnp.float32)

    @pl.when(f == 0)
    def _():
        acc_ref[...] = part

    @pl.when(f > 0)
    def _():
        acc_ref[...] = acc_ref[...] + part

    @pl.when(f == NF - 1)
    def _():
        w = wt_ref[0][:, :1]
        out_ref[0] = (acc_ref[...] + b2_ref[0]) * w


def _mlp(xs3, W1, b1, W2, b2, wts_b):
    return pl.pallas_call(
        _mlp_body,
        grid=(E, NF),
        in_specs=[
            pl.BlockSpec((1, K, D), lambda e, f: (e, 0, 0)),
            pl.BlockSpec((1, F, D), lambda e, f: (e, f, 0)),
            pl.BlockSpec((1, 1, F), lambda e, f: (e, 0, f)),
            pl.BlockSpec((1, D, F), lambda e, f: (e, 0, f)),
            pl.BlockSpec((1, 1, D), lambda e, f: (e, 0, 0)),
            pl.BlockSpec((1, K, 128), lambda e, f: (e, 0, 0)),
        ],
        out_specs=pl.BlockSpec((1, K, D), lambda e, f: (e, 0, 0)),
        out_shape=jax.ShapeDtypeStruct((E, K, D), jnp.float32),
        scratch_shapes=[pltpu.VMEM((K, D), jnp.float32)],
    )(xs3, W1, b1[:, None, :], W2, b2[:, None, :], wts_b)


NC = 2
NS = 16
NW = NC * NS
GROWS = T // NW
GCHUNK = 32
GNCH = GROWS // GCHUNK


@functools.lru_cache(maxsize=None)
def _make_sc_gather():
    @functools.partial(
        pl.kernel,
        out_type=jax.ShapeDtypeStruct((T, D), jnp.float32),
        mesh=plsc.VectorSubcoreMesh(core_axis_name="c", subcore_axis_name="s",
                                    num_cores=NC, num_subcores=NS),
        scratch_types=[
            pltpu.VMEM((GCHUNK,), jnp.int32),
            pltpu.VMEM((GCHUNK, D), jnp.float32),
            pltpu.SemaphoreType.DMA,
        ],
        compiler_params=pltpu.CompilerParams(needs_layout_passes=False),
    )
    def _sc_gather(x_hbm, sel_hbm, out_hbm, idx_v, rows_v, sem):
        wid = lax.axis_index("s") * NC + lax.axis_index("c")
        base = wid * GROWS
        for c in range(GNCH):
            off = base + c * GCHUNK
            pltpu.sync_copy(sel_hbm.at[pl.ds(off, GCHUNK)], idx_v)
            pltpu.async_copy(x_hbm.at[idx_v], rows_v, sem).wait()
            pltpu.sync_copy(rows_v, out_hbm.at[pl.ds(off, GCHUNK)])

    return _sc_gather


TOK_B = 64
NBUK = T // TOK_B
CAPM = 544
SPAD = T + 32


@functools.lru_cache(maxsize=None)
def _make_sc_scatter():
    @functools.partial(
        pl.kernel,
        out_type=jax.ShapeDtypeStruct((T, D), jnp.float32),
        mesh=plsc.VectorSubcoreMesh(core_axis_name="c", subcore_axis_name="s",
                                    num_cores=NC, num_subcores=NS),
        scratch_types=[
            pltpu.VMEM((T,), jnp.int32),
            pltpu.VMEM((CAPM,), jnp.int32),
            pltpu.VMEM((CAPM,), jnp.int32),
            pltpu.VMEM((CAPM // 32, 32), jnp.int32),
            pltpu.VMEM((CAPM // 32, 32), jnp.int32),
            pltpu.VMEM((32, D), jnp.float32),
            pltpu.VMEM((TOK_B, D), jnp.float32),
            pltpu.SemaphoreType.DMA,
        ],
        compiler_params=pltpu.CompilerParams(needs_layout_passes=False),
    )
    def _sc_scatter(src_hbm, sel_hbm, zeros_hbm, out_hbm,
                    sel_v, mlist, llist, m2, l2, buf, acc, sem):
        wid = lax.axis_index("s") * NC + lax.axis_index("c")
        pltpu.sync_copy(sel_hbm, sel_v)
        dummy_slot = jnp.full((16,), T, jnp.int32)
        zero16 = jnp.zeros((16,), jnp.int32)
        for p in range(NBUK // NW):
            tb = (p * NW + wid) * TOK_B
            pltpu.sync_copy(zeros_hbm, acc)

            def fill(i, _):
                mlist[pl.ds(i * 16, 16)] = dummy_slot
                llist[pl.ds(i * 16, 16)] = zero16
                return 0

            lax.fori_loop(0, CAPM // 16, fill, 0)

            lane16 = lax.iota(jnp.int32, 16)

            def scan(g, cnt):
                tk = sel_v[pl.ds(g * 16, 16)]
                loc = tk - tb
                m = (loc >= 0) & (loc < TOK_B)
                slots = lane16 + g * 16
                ones = jnp.where(m, 1, 0)
                pos = cnt + plsc.cumsum(ones) - 1
                pos = jnp.where(m, pos, CAPM - 16 + lane16)
                plsc.store_scatter(mlist, [pos], slots)
                plsc.store_scatter(llist, [pos], jnp.where(m, loc, 0))
                return cnt + jnp.sum(ones)

            cnt = lax.fori_loop(0, T // 16, scan, jnp.int32(0))
            nch = (cnt + 31) // 32

            for c in range(CAPM // 32):
                m2[c, pl.ds(0, 16)] = mlist[pl.ds(c * 32, 16)]
                m2[c, pl.ds(16, 16)] = mlist[pl.ds(c * 32 + 16, 16)]
                l2[c, pl.ds(0, 16)] = llist[pl.ds(c * 32, 16)]
                l2[c, pl.ds(16, 16)] = llist[pl.ds(c * 32 + 16, 16)]

            def gath(c, _):
                pltpu.async_copy(src_hbm.at[m2.at[c]], buf, sem).wait()
                nrow = jnp.minimum(cnt - c * 32, 32)

                def row(r, _):
                    lvec = l2[c, pl.ds((r // 16) * 16, 16)]
                    lr = jnp.sum(jnp.where(lane16 == (r % 16), lvec, 0))

                    def addrow(cc, _):
                        b = cc * 64
                        for u in range(4):
                            plsc.addupdate(acc.at[lr, pl.ds(b + u * 16, 16)],
                                           buf[r, pl.ds(b + u * 16, 16)])
                        return 0

                    lax.fori_loop(0, D // 64, addrow, 0)
                    return 0

                lax.fori_loop(0, nrow, row, 0)
                return 0

            lax.fori_loop(0, nch, gath, 0)
            pltpu.sync_copy(acc, out_hbm.at[pl.ds(tb, TOK_B)])

    return _sc_scatter


def kernel(inputs, Wr, W1, b1, W2, b2):
    x2d = inputs.reshape(-1, D)

    router_logits_x = x2d @ Wr.T
    probs_t = jax.nn.softmax(router_logits_x.astype(jnp.float32), axis=1).T

    wr_pad = jnp.zeros((EPAD, D), jnp.float32).at[:E].set(Wr)
    logits_pad, wts, sel = _router_topk(x2d, wr_pad, probs_t)
    router_logits = logits_pad[:, :E]

    sel_flat = sel.reshape(T)
    xs = _make_sc_gather()(x2d, sel_flat)

    wts_b = jnp.broadcast_to(wts[:, :, None], (E, K, 128))
    scaled = _mlp(xs.reshape(E, K, D), W1, b1, W2, b2, wts_b)

    zeros_hbm = jnp.zeros((TOK_B, D), jnp.float32)
    scaled_pad = jnp.concatenate(
        [scaled.reshape(T, D), jnp.zeros((SPAD - T, D), jnp.float32)])
    results = _make_sc_scatter()(scaled_pad, sel_flat, zeros_hbm)
    return results.reshape(inputs.shape), router_logits, sel


# speedup vs baseline: 1.5226x; 1.0458x over previous
import functools

import jax
import jax.numpy as jnp
from jax import lax
from jax.experimental import pallas as pl
from jax.experimental.pallas import tpu as pltpu
from jax.experimental.pallas import tpu_sc as plsc

E = 8
D = 1024
DFF = 4096
T = 4096
K = 512
EPAD = 128


def _router_topk_body(x_ref, wr_ref, pt_ref, logits_ref, wts_ref, sel_ref):
    logits_ref[...] = lax.dot_general(
        x_ref[...], wr_ref[...], (((1,), (1,)), ((), ())),
        preferred_element_type=jnp.float32)

    v = pt_ref[...]
    idx = lax.broadcasted_iota(jnp.int32, (E, T), 1)
    iot = idx

    k = 2
    while k <= T:
        j = k // 2
        while j >= 1:
            bit = (iot & j) != 0
            pv = jnp.where(bit, jnp.roll(v, j, 1), jnp.roll(v, -j, 1))
            pi = jnp.where(bit, jnp.roll(idx, j, 1), jnp.roll(idx, -j, 1))
            less = (v > pv) | ((v == pv) & (idx < pi))
            want_min = ((iot & k) == 0) ^ bit
            keep = want_min == less
            v = jnp.where(keep, v, pv)
            idx = jnp.where(keep, idx, pi)
            j //= 2
        k *= 2

    wts_ref[...] = v[:, :K]
    sel_ref[...] = idx[:, :K]


def _router_topk(x2d, wr_pad, probs_t):
    return pl.pallas_call(
        _router_topk_body,
        out_shape=(
            jax.ShapeDtypeStruct((T, EPAD), jnp.float32),
            jax.ShapeDtypeStruct((E, K), jnp.float32),
            jax.ShapeDtypeStruct((E, K), jnp.int32),
        ),
    )(x2d, wr_pad, probs_t)


NF = 4
F = DFF // NF


def _erf(x):
    s = jnp.sign(x)
    a = jnp.abs(x)
    t = 1.0 / (1.0 + 0.3275911 * a)
    p = ((((1.061405429 * t - 1.453152027) * t + 1.421413741) * t
          - 0.284496736) * t + 0.254829592) * t
    return s * (1.0 - p * jnp.exp(-a * a))


def _gelu(x):
    return 0.5 * x * (1.0 + _erf(x * 0.7071067811865476))


def _mlp_body(xs_ref, w1_ref, b1_ref, w2_ref, b2_ref, wt_ref, out_ref, acc_ref):
    f = pl.program_id(1)
    x2 = xs_ref[0].astype(jnp.bfloat16)
    h = lax.dot_general(x2, w1_ref[0].astype(jnp.bfloat16),
                        (((1,), (1,)), ((), ())),
                        preferred_element_type=jnp.float32)
    h = _gelu(h + b1_ref[0])
    part = lax.dot_general(h.astype(jnp.bfloat16), w2_ref[0].astype(jnp.bfloat16),
                           (((1,), (1,)), ((), ())),
                           preferred_element_type=jnp.float32)

    @pl.when(f == 0)
    def _():
        acc_ref[...] = part

    @pl.when(f > 0)
    def _():
        acc_ref[...] = acc_ref[...] + part

    @pl.when(f == NF - 1)
    def _():
        w = wt_ref[0][:, :1]
        out_ref[0] = (acc_ref[...] + b2_ref[0]) * w


def _mlp(xs3, W1, b1, W2, b2, wts_b):
    return pl.pallas_call(
        _mlp_body,
        grid=(E, NF),
        in_specs=[
            pl.BlockSpec((1, K, D), lambda e, f: (e, 0, 0)),
            pl.BlockSpec((1, F, D), lambda e, f: (e, f, 0)),
            pl.BlockSpec((1, 1, F), lambda e, f: (e, 0, f)),
            pl.BlockSpec((1, D, F), lambda e, f: (e, 0, f)),
            pl.BlockSpec((1, 1, D), lambda e, f: (e, 0, 0)),
            pl.BlockSpec((1, K, 128), lambda e, f: (e, 0, 0)),
        ],
        out_specs=pl.BlockSpec((1, K, D), lambda e, f: (e, 0, 0)),
        out_shape=jax.ShapeDtypeStruct((E, K, D), jnp.float32),
        scratch_shapes=[pltpu.VMEM((K, D), jnp.float32)],
    )(xs3, W1, b1[:, None, :], W2, b2[:, None, :], wts_b)


NC = 2
NS = 16
NW = NC * NS
GROWS = T // NW
GCHUNK = 32
GNCH = GROWS // GCHUNK


@functools.lru_cache(maxsize=None)
def _make_sc_gather():
    @functools.partial(
        pl.kernel,
        out_type=jax.ShapeDtypeStruct((T, D), jnp.float32),
        mesh=plsc.VectorSubcoreMesh(core_axis_name="c", subcore_axis_name="s",
                                    num_cores=NC, num_subcores=NS),
        scratch_types=[
            pltpu.VMEM((GCHUNK,), jnp.int32),
            pltpu.VMEM((GCHUNK, D), jnp.float32),
            pltpu.SemaphoreType.DMA,
        ],
        compiler_params=pltpu.CompilerParams(needs_layout_passes=False),
    )
    def _sc_gather(x_hbm, sel_hbm, out_hbm, idx_v, rows_v, sem):
        wid = lax.axis_index("s") * NC + lax.axis_index("c")
        base = wid * GROWS
        for c in range(GNCH):
            off = base + c * GCHUNK
            pltpu.sync_copy(sel_hbm.at[pl.ds(off, GCHUNK)], idx_v)
            pltpu.async_copy(x_hbm.at[idx_v], rows_v, sem).wait()
            pltpu.sync_copy(rows_v, out_hbm.at[pl.ds(off, GCHUNK)])

    return _sc_gather


TOK_B = 64
NBUK = T // TOK_B
CAPM = 544


@functools.lru_cache(maxsize=None)
def _make_sc_scatter():
    @functools.partial(
        pl.kernel,
        out_type=jax.ShapeDtypeStruct((T, D), jnp.float32),
        mesh=plsc.VectorSubcoreMesh(core_axis_name="c", subcore_axis_name="s",
                                    num_cores=NC, num_subcores=NS),
        scratch_types=[
            pltpu.VMEM((T,), jnp.int32),
            pltpu.VMEM((CAPM,), jnp.int32),
            pltpu.VMEM((CAPM,), jnp.int32),
            pltpu.VMEM((CAPM // 32, 32), jnp.int32),
            pltpu.VMEM((CAPM // 32, 32), jnp.int32),
            pltpu.VMEM((32, D), jnp.float32),
            pltpu.VMEM((TOK_B, D), jnp.float32),
            pltpu.SemaphoreType.DMA,
        ],
        compiler_params=pltpu.CompilerParams(needs_layout_passes=False),
    )
    def _sc_scatter(src_hbm, sel_hbm, zeros_hbm, out_hbm,
                    sel_v, mlist, llist, m2, l2, buf, acc, sem):
        wid = lax.axis_index("s") * NC + lax.axis_index("c")
        pltpu.sync_copy(sel_hbm, sel_v)
        dummy_slot = jnp.zeros((16,), jnp.int32)
        zero16 = jnp.zeros((16,), jnp.int32)
        for p in range(NBUK // NW):
            tb = (p * NW + wid) * TOK_B
            pltpu.sync_copy(zeros_hbm, acc)

            def fill(i, _):
                mlist[pl.ds(i * 16, 16)] = dummy_slot
                llist[pl.ds(i * 16, 16)] = zero16
                return 0

            lax.fori_loop(0, CAPM // 16, fill, 0)

            lane16 = lax.iota(jnp.int32, 16)

            def scan(g, cnt):
                tk = sel_v[pl.ds(g * 16, 16)]
                loc = tk - tb
                m = (loc >= 0) & (loc < TOK_B)
                slots = lane16 + g * 16
                ones = jnp.where(m, 1, 0)
                pos = cnt + plsc.cumsum(ones) - 1
                pos = jnp.where(m, pos, CAPM - 16 + lane16)
                plsc.store_scatter(mlist, [pos], slots)
                plsc.store_scatter(llist, [pos], jnp.where(m, loc, 0))
                return cnt + jnp.sum(ones)

            cnt = lax.fori_loop(0, T // 16, scan, jnp.int32(0))
            nch = (cnt + 31) // 32

            for c in range(CAPM // 32):
                m2[c, pl.ds(0, 16)] = mlist[pl.ds(c * 32, 16)]
                m2[c, pl.ds(16, 16)] = mlist[pl.ds(c * 32 + 16, 16)]
                l2[c, pl.ds(0, 16)] = llist[pl.ds(c * 32, 16)]
                l2[c, pl.ds(16, 16)] = llist[pl.ds(c * 32 + 16, 16)]

            def gath(c, _):
                pltpu.async_copy(src_hbm.at[m2.at[c]], buf, sem).wait()
                nrow = jnp.minimum(cnt - c * 32, 32)

                def row(r, _):
                    lvec = l2[c, pl.ds((r // 16) * 16, 16)]
                    lr = jnp.sum(jnp.where(lane16 == (r % 16), lvec, 0))

                    def addrow(cc, _):
                        b = cc * 64
                        for u in range(4):
                            plsc.addupdate(acc.at[lr, pl.ds(b + u * 16, 16)],
                                           buf[r, pl.ds(b + u * 16, 16)])
                        return 0

                    lax.fori_loop(0, D // 64, addrow, 0)
                    return 0

                lax.fori_loop(0, nrow, row, 0)
                return 0

            lax.fori_loop(0, nch, gath, 0)
            pltpu.sync_copy(acc, out_hbm.at[pl.ds(tb, TOK_B)])

    return _sc_scatter


def kernel(inputs, Wr, W1, b1, W2, b2):
    x2d = inputs.reshape(-1, D)

    router_logits_x = x2d @ Wr.T
    probs_t = jax.nn.softmax(router_logits_x.astype(jnp.float32), axis=1).T

    wr_pad = jnp.zeros((EPAD, D), jnp.float32).at[:E].set(Wr)
    logits_pad, wts, sel = _router_topk(x2d, wr_pad, probs_t)
    router_logits = logits_pad[:, :E]

    sel_flat = sel.reshape(T)
    xs = _make_sc_gather()(x2d, sel_flat)

    wts_b = jnp.broadcast_to(wts[:, :, None], (E, K, 128))
    scaled = _mlp(xs.reshape(E, K, D), W1, b1, W2, b2, wts_b)

    zeros_hbm = jnp.zeros((TOK_B, D), jnp.float32)
    results = _make_sc_scatter()(scaled.reshape(T, D), sel_flat, zeros_hbm)
    return results.reshape(inputs.shape), router_logits, sel
